# baseline (speedup 1.0000x reference)
import jax
import jax.numpy as jnp
from jax.experimental import pallas as pl
from jax.experimental.pallas import tpu as pltpu


_BLOCK_S = 512


def _body(x_ref, p_ref, o_ref):
    o_ref[...] = x_ref[...] + p_ref[...][None]


def kernel(x, pos_table):
    B, S, D = x.shape
    bs = min(_BLOCK_S, S)
    grid = (S // bs,)
    return pl.pallas_call(
        _body,
        grid=grid,
        in_specs=[
            pl.BlockSpec((B, bs, D), lambda s: (0, s, 0)),
            pl.BlockSpec((bs, D), lambda s: (s, 0)),
        ],
        out_specs=pl.BlockSpec((B, bs, D), lambda s: (0, s, 0)),
        out_shape=jax.ShapeDtypeStruct(x.shape, x.dtype),
        compiler_params=pltpu.CompilerParams(
            dimension_semantics=("parallel",),
        ),
    )(x, pos_table)
